# Initial kernel scaffold; baseline (speedup 1.0000x reference)
import jax
import jax.numpy as jnp
from jax.experimental import pallas as pl


def kernel(x, edge_index, edge_weight, W10, b10, W11, b11, W12, b12, g1, bb1, W20, b20, W21, b21, W22, b22, Wf, bf):
    raise NotImplementedError("write your pallas kernel here")


# speedup vs baseline: 6.3411x; 6.3411x over previous
import functools

import jax
import jax.numpy as jnp
from jax import lax
from jax.experimental import pallas as pl
from jax.experimental.pallas import tpu as pltpu
from jax.experimental.pallas import tpu_sc as plsc

N = 10000
E = 320000
D = 128
EP = 331776
PAD = EP - (E + N)
EPT = EP // 32
NCH = EPT // 128
EPT16 = EP // 16
NCH16 = EPT16 // 128

_MESH = plsc.VectorSubcoreMesh(
    core_axis_name="c", subcore_axis_name="s", num_cores=2, num_subcores=16)


def _norm_body(row_h, col_h, w_h, norm_h, dis_h,
               deg_sh, zb, ci, wb, ri, nb, deg_v, dis_v):
    cid = lax.axis_index("c")
    sid = lax.axis_index("s")
    wid = cid * 16 + sid

    def _z(i, _):
        zb[pl.ds(i * 16, 16)] = jnp.zeros((16,), jnp.float32)
        return 0
    lax.fori_loop(0, 40, _z, 0)

    @pl.when(sid < 15)
    def _():
        pltpu.sync_copy(zb, deg_sh.at[pl.ds(sid * 640, 640)])

    @pl.when(sid == 15)
    def _():
        pltpu.sync_copy(zb.at[pl.ds(0, 400)], deg_sh.at[pl.ds(9600, 400)])

    plsc.subcore_barrier()

    def _deg_chunk(c, _):
        base = sid * EPT16 + c * 128
        pltpu.sync_copy(col_h.at[pl.ds(base, 128)], ci)
        pltpu.sync_copy(w_h.at[pl.ds(base, 128)], wb)
        pltpu.sync_copy(wb, deg_sh.at[ci], add=True)
        return 0
    lax.fori_loop(0, NCH16, _deg_chunk, 0)

    plsc.subcore_barrier()

    pltpu.sync_copy(deg_sh, deg_v)
    magic = jnp.int32(0x5F3759DF)

    def _rsqrt(i, _):
        d = deg_v[pl.ds(i * 16, 16)]
        yi = magic - lax.shift_right_logical(
            lax.bitcast_convert_type(d, jnp.int32), 1)
        y = lax.bitcast_convert_type(yi, jnp.float32)
        h = d * 0.5
        for _ in range(3):
            y = y * (1.5 - h * y * y)
        dis_v[pl.ds(i * 16, 16)] = y
        return 0
    lax.fori_loop(0, 625, _rsqrt, 0)

    def _norm_chunk(c, _):
        base = wid * EPT + c * 128
        pltpu.sync_copy(row_h.at[pl.ds(base, 128)], ri)
        pltpu.sync_copy(col_h.at[pl.ds(base, 128)], ci)
        pltpu.sync_copy(w_h.at[pl.ds(base, 128)], wb)

        def _g(g, _):
            r = ri[pl.ds(g * 16, 16)]
            cc = ci[pl.ds(g * 16, 16)]
            nr = (plsc.load_gather(dis_v, [r]) * wb[pl.ds(g * 16, 16)]
                  * plsc.load_gather(dis_v, [cc]))
            nb[pl.ds(g * 16, 16)] = nr
            return 0
        lax.fori_loop(0, 8, _g, 0)
        pltpu.sync_copy(nb, norm_h.at[pl.ds(base, 128)])
        return 0
    lax.fori_loop(0, NCH, _norm_chunk, 0)

    @pl.when(cid == 0)
    def _():
        @pl.when(sid < 15)
        def _():
            pltpu.sync_copy(dis_v.at[pl.ds(sid * 640, 640)],
                            dis_h.at[pl.ds(sid * 640, 640)])

        @pl.when(sid == 15)
        def _():
            pltpu.sync_copy(dis_v.at[pl.ds(9600, 400)],
                            dis_h.at[pl.ds(9600, 400)])


_norm_call = pl.kernel(
    _norm_body,
    out_type=(jax.ShapeDtypeStruct((EP,), jnp.float32),
              jax.ShapeDtypeStruct((N,), jnp.float32)),
    mesh=_MESH,
    scratch_types=[
        pltpu.VMEM_SHARED((N,), jnp.float32),
        pltpu.VMEM((640,), jnp.float32),
        pltpu.VMEM((128,), jnp.int32),
        pltpu.VMEM((128,), jnp.float32),
        pltpu.VMEM((128,), jnp.int32),
        pltpu.VMEM((128,), jnp.float32),
        pltpu.VMEM((N,), jnp.float32),
        pltpu.VMEM((N,), jnp.float32),
    ],
    compiler_params=pltpu.CompilerParams(needs_layout_passes=False),
    name="mixhop_norm_sc",
)


def _spmm_body(v_h, row_h, col_h, norm_h, out_h,
               acc_sh, zb, ri, ci, nb, rows, sem):
    cid = lax.axis_index("c")
    sid = lax.axis_index("s")
    wid = cid * 16 + sid

    def _z(i, _):
        r = lax.div(i, 8)
        jc = lax.rem(i, 8) * 16
        zb[r, pl.ds(jc, 16)] = jnp.zeros((16,), jnp.float32)
        return 0
    lax.fori_loop(0, 512, _z, 0)

    @pl.when(sid < 15)
    def _():
        def _za(k, _):
            pltpu.sync_copy(zb, acc_sh.at[pl.ds(sid * 640 + k * 64, 64), :])
            return 0
        lax.fori_loop(0, 10, _za, 0)

    @pl.when(sid == 15)
    def _():
        def _za(k, _):
            pltpu.sync_copy(zb, acc_sh.at[pl.ds(9600 + k * 64, 64), :])
            return 0
        lax.fori_loop(0, 6, _za, 0)
        pltpu.sync_copy(zb.at[pl.ds(0, 16), :], acc_sh.at[pl.ds(9984, 16), :])

    plsc.subcore_barrier()

    def _chunk(c, _):
        base = wid * EPT + c * 128
        pltpu.sync_copy(row_h.at[pl.ds(base, 128)], ri)
        pltpu.sync_copy(col_h.at[pl.ds(base, 128)], ci)
        pltpu.sync_copy(norm_h.at[pl.ds(base, 128)], nb)
        pltpu.async_copy(v_h.at[ri], rows, sem).wait()

        def _scale(g, _):
            n16 = nb[pl.ds(g * 16, 16)]
            for k in range(16):
                e = g * 16 + k
                s = n16[k]
                for j in range(8):
                    rows[e, pl.ds(j * 16, 16)] = (
                        rows[e, pl.ds(j * 16, 16)] * s)
            return 0
        lax.fori_loop(0, 8, _scale, 0)
        pltpu.sync_copy(rows, acc_sh.at[ci], add=True)
        return 0
    lax.fori_loop(0, NCH, _chunk, 0)

    plsc.subcore_barrier()

    @pl.when(sid < 15)
    def _():
        pltpu.sync_copy(acc_sh.at[pl.ds(sid * 640, 640), :],
                        out_h.at[cid, pl.ds(sid * 640, 640), :])

    @pl.when(sid == 15)
    def _():
        pltpu.sync_copy(acc_sh.at[pl.ds(9600, 400), :],
                        out_h.at[cid, pl.ds(9600, 400), :])


_spmm_call = pl.kernel(
    _spmm_body,
    out_type=jax.ShapeDtypeStruct((2, N, D), jnp.float32),
    mesh=_MESH,
    scratch_types=[
        pltpu.VMEM_SHARED((N, D), jnp.float32),
        pltpu.VMEM((64, D), jnp.float32),
        pltpu.VMEM((128,), jnp.int32),
        pltpu.VMEM((128,), jnp.int32),
        pltpu.VMEM((128,), jnp.float32),
        pltpu.VMEM((128, D), jnp.float32),
        pltpu.SemaphoreType.DMA,
    ],
    compiler_params=pltpu.CompilerParams(needs_layout_passes=False),
    name="mixhop_spmm_sc",
)


_BR = 1000
_GRID = N // _BR


def _dot_t(a, w):
    return lax.dot_general(a, w, (((1,), (1,)), ((), ())),
                           preferred_element_type=jnp.float32)


def _lin3_tc(x_ref, w0, w1, w2, b0, b1, b2, o0, o1, o2):
    x = x_ref[...]
    o0[...] = _dot_t(x, w0[...]) + b0[...]
    o1[...] = _dot_t(x, w1[...]) + b1[...]
    o2[...] = _dot_t(x, w2[...]) + b2[...]


def _add2_tc(a, b, o):
    o[...] = a[...] + b[...]


def _lin3b_tc(h00, p1a, p1b, p3a, p3b, g, bb, w0, w1, w2, b0, b1, b2,
              o0, o1, o2):
    hcat = jnp.concatenate(
        [h00[...], p1a[...] + p1b[...], p3a[...] + p3b[...]], axis=1)
    z = jnp.maximum(hcat * g[...] + bb[...], 0.0)
    o0[...] = _dot_t(z, w0[...]) + b0[...]
    o1[...] = _dot_t(z, w1[...]) + b1[...]
    o2[...] = _dot_t(z, w2[...]) + b2[...]


def _final_tc(t20, q1a, q1b, q3a, q3b, wf, bf, o):
    hcat = jnp.concatenate(
        [t20[...], q1a[...] + q1b[...], q3a[...] + q3b[...]], axis=1)
    o[...] = _dot_t(hcat, wf[...]) + bf[...]


def _row_spec(d):
    return pl.BlockSpec((_BR, d), lambda i: (i, 0))


def _full_spec(shape):
    return pl.BlockSpec(shape, lambda i: (0,) * len(shape))


def _lin3(x, w0, b0, w1, b1, w2, b2):
    outs = [jax.ShapeDtypeStruct((N, D), jnp.float32)] * 3
    din = x.shape[1]
    return pl.pallas_call(
        _lin3_tc,
        grid=(_GRID,),
        in_specs=[_row_spec(din)] + [_full_spec((D, din))] * 3
        + [_full_spec((1, D))] * 3,
        out_specs=[_row_spec(D)] * 3,
        out_shape=outs,
    )(x, w0, w1, w2, b0.reshape(1, D), b1.reshape(1, D), b2.reshape(1, D))


def _add2(a, b):
    return pl.pallas_call(
        _add2_tc,
        grid=(_GRID,),
        in_specs=[_row_spec(D)] * 2,
        out_specs=_row_spec(D),
        out_shape=jax.ShapeDtypeStruct((N, D), jnp.float32),
    )(a, b)


def _lin3b(h00, p1a, p1b, p3a, p3b, g, bb, w0, b0, w1, b1, w2, b2):
    outs = [jax.ShapeDtypeStruct((N, D), jnp.float32)] * 3
    return pl.pallas_call(
        _lin3b_tc,
        grid=(_GRID,),
        in_specs=[_row_spec(D)] * 5 + [_full_spec((1, 3 * D))] * 2
        + [_full_spec((D, 3 * D))] * 3 + [_full_spec((1, D))] * 3,
        out_specs=[_row_spec(D)] * 3,
        out_shape=outs,
    )(h00, p1a, p1b, p3a, p3b, g.reshape(1, 3 * D), bb.reshape(1, 3 * D),
      w0, w1, w2, b0.reshape(1, D), b1.reshape(1, D), b2.reshape(1, D))


def _final(t20, q1a, q1b, q3a, q3b, wf, bf):
    return pl.pallas_call(
        _final_tc,
        grid=(_GRID,),
        in_specs=[_row_spec(D)] * 5 + [_full_spec((D, 3 * D)),
                                       _full_spec((1, D))],
        out_specs=_row_spec(D),
        out_shape=jax.ShapeDtypeStruct((N, D), jnp.float32),
    )(t20, q1a, q1b, q3a, q3b, wf, bf.reshape(1, D))


def kernel(x, edge_index, edge_weight, W10, b10, W11, b11, W12, b12, g1, bb1,
           W20, b20, W21, b21, W22, b22, Wf, bf):
    loop = jnp.arange(N, dtype=jnp.int32)
    zpad = jnp.zeros((PAD,), jnp.int32)
    row_ext = jnp.concatenate([edge_index[0].astype(jnp.int32), loop, zpad])
    col_ext = jnp.concatenate([edge_index[1].astype(jnp.int32), loop, zpad])
    w_ext = jnp.concatenate([edge_weight.astype(jnp.float32),
                             jnp.ones((N,), jnp.float32),
                             jnp.zeros((PAD,), jnp.float32)])

    norm, _dis = _norm_call(row_ext, col_ext, w_ext)

    h00, t1, t2 = _lin3(x, W10, b10, W11, b11, W12, b12)
    p1 = _spmm_call(t1, row_ext, col_ext, norm)
    p2 = _spmm_call(t2, row_ext, col_ext, norm)
    u = _add2(p2[0], p2[1])
    p3 = _spmm_call(u, row_ext, col_ext, norm)

    t20, t21, t22 = _lin3b(h00, p1[0], p1[1], p3[0], p3[1],
                           g1, bb1, W20, b20, W21, b21, W22, b22)
    q1 = _spmm_call(t21, row_ext, col_ext, norm)
    q2 = _spmm_call(t22, row_ext, col_ext, norm)
    v2 = _add2(q2[0], q2[1])
    q3 = _spmm_call(v2, row_ext, col_ext, norm)

    return _final(t20, q1[0], q1[1], q3[0], q3[1], Wf, bf)


# speedup vs baseline: 12.3543x; 1.9483x over previous
import functools

import jax
import jax.numpy as jnp
from jax import lax
from jax.experimental import pallas as pl
from jax.experimental.pallas import tpu as pltpu
from jax.experimental.pallas import tpu_sc as plsc

N = 10000
E = 320000
D = 128
EP = 331776
PAD = EP - (E + N)
EPT = EP // 32
NCH = EPT // 128
EPT16 = EP // 16
NCH16 = EPT16 // 128

_MESH = plsc.VectorSubcoreMesh(
    core_axis_name="c", subcore_axis_name="s", num_cores=2, num_subcores=16)


def _norm_body(row_h, col_h, w_h, norm_h, dis_h,
               deg_sh, zb, colb, wbb, rowb, outb,
               cs0, cs1, ws0, ws1, deg_v, dis_v, sem0, sem1):
    cid = lax.axis_index("c")
    sid = lax.axis_index("s")
    wid = cid * 16 + sid

    def _z(i, _):
        zb[pl.ds(i * 16, 16)] = jnp.zeros((16,), jnp.float32)
        return 0
    lax.fori_loop(0, 40, _z, 0)

    @pl.when(sid < 15)
    def _():
        pltpu.sync_copy(zb, deg_sh.at[pl.ds(sid * 640, 640)])

    @pl.when(sid == 15)
    def _():
        pltpu.sync_copy(zb.at[pl.ds(0, 400)], deg_sh.at[pl.ds(9600, 400)])

    plsc.subcore_barrier()

    def _deg_stage(c, cs, ws, sem):
        @pl.when(lax.rem(c, 27) == 0)
        def _():
            base = sid * EPT16 + lax.div(c, 27) * 3456
            pltpu.sync_copy(col_h.at[pl.ds(base, 3456)], colb)
            pltpu.sync_copy(w_h.at[pl.ds(base, 3456)], wbb)

        @pl.when(c >= 2)
        def _():
            pltpu.make_async_copy(ws, deg_sh.at[cs], sem).wait()

        k = lax.rem(c, 27)

        def _cp(g, _):
            off = k * 128 + g * 16
            cs[pl.ds(g * 16, 16)] = colb[pl.ds(off, 16)]
            ws[pl.ds(g * 16, 16)] = wbb[pl.ds(off, 16)]
            return 0
        lax.fori_loop(0, 8, _cp, 0)
        pltpu.async_copy(ws, deg_sh.at[cs], sem, add=True)

    def _deg_iter(i, _):
        _deg_stage(2 * i, cs0, ws0, sem0)
        _deg_stage(2 * i + 1, cs1, ws1, sem1)
        return 0
    lax.fori_loop(0, NCH16 // 2, _deg_iter, 0)
    pltpu.make_async_copy(ws0, deg_sh.at[cs0], sem0).wait()
    pltpu.make_async_copy(ws1, deg_sh.at[cs1], sem1).wait()

    plsc.subcore_barrier()

    pltpu.sync_copy(deg_sh, deg_v)
    magic = jnp.int32(0x5F3759DF)

    def _rsqrt(i, _):
        d = deg_v[pl.ds(i * 16, 16)]
        yi = magic - lax.shift_right_logical(
            lax.bitcast_convert_type(d, jnp.int32), 1)
        y = lax.bitcast_convert_type(yi, jnp.float32)
        h = d * 0.5
        for _ in range(3):
            y = y * (1.5 - h * y * y)
        dis_v[pl.ds(i * 16, 16)] = y
        return 0
    lax.fori_loop(0, 625, _rsqrt, 0)

    for blk in range(3):
        base = wid * EPT + blk * 3456
        pltpu.sync_copy(row_h.at[pl.ds(base, 3456)], rowb)
        pltpu.sync_copy(col_h.at[pl.ds(base, 3456)], colb)
        pltpu.sync_copy(w_h.at[pl.ds(base, 3456)], wbb)

        def _g(g, _):
            r = rowb[pl.ds(g * 16, 16)]
            cc = colb[pl.ds(g * 16, 16)]
            nr = (plsc.load_gather(dis_v, [r]) * wbb[pl.ds(g * 16, 16)]
                  * plsc.load_gather(dis_v, [cc]))
            outb[pl.ds(g * 16, 16)] = nr
            return 0
        lax.fori_loop(0, 216, _g, 0)
        pltpu.sync_copy(outb, norm_h.at[pl.ds(base, 3456)])

    @pl.when(cid == 0)
    def _():
        @pl.when(sid < 15)
        def _():
            pltpu.sync_copy(dis_v.at[pl.ds(sid * 640, 640)],
                            dis_h.at[pl.ds(sid * 640, 640)])

        @pl.when(sid == 15)
        def _():
            pltpu.sync_copy(dis_v.at[pl.ds(9600, 400)],
                            dis_h.at[pl.ds(9600, 400)])


_norm_call = pl.kernel(
    _norm_body,
    out_type=(jax.ShapeDtypeStruct((EP,), jnp.float32),
              jax.ShapeDtypeStruct((N,), jnp.float32)),
    mesh=_MESH,
    scratch_types=[
        pltpu.VMEM_SHARED((N,), jnp.float32),
        pltpu.VMEM((640,), jnp.float32),
        pltpu.VMEM((3456,), jnp.int32),
        pltpu.VMEM((3456,), jnp.float32),
        pltpu.VMEM((3456,), jnp.int32),
        pltpu.VMEM((3456,), jnp.float32),
        pltpu.VMEM((128,), jnp.int32),
        pltpu.VMEM((128,), jnp.int32),
        pltpu.VMEM((128,), jnp.float32),
        pltpu.VMEM((128,), jnp.float32),
        pltpu.VMEM((N,), jnp.float32),
        pltpu.VMEM((N,), jnp.float32),
        pltpu.SemaphoreType.DMA,
        pltpu.SemaphoreType.DMA,
    ],
    compiler_params=pltpu.CompilerParams(needs_layout_passes=False),
    name="mixhop_norm_sc",
)


def _spmm_body(v_h, row_h, col_h, norm_h, out_h,
               acc_sh, zb, rowb, colb, nrmb,
               rows0, rows1, rs0, rs1, cs0, cs1, ns0, ns1,
               gsem0, gsem1, ssem0, ssem1):
    cid = lax.axis_index("c")
    sid = lax.axis_index("s")
    wid = cid * 16 + sid

    def _z(i, _):
        zb[lax.div(i, 8), pl.ds(lax.rem(i, 8) * 16, 16)] = (
            jnp.zeros((16,), jnp.float32))
        return 0
    lax.fori_loop(0, 256, _z, 0)

    @pl.when(sid < 15)
    def _():
        def _za(k, _):
            pltpu.sync_copy(zb, acc_sh.at[pl.ds(sid * 640 + k * 32, 32), :])
            return 0
        lax.fori_loop(0, 20, _za, 0)

    @pl.when(sid == 15)
    def _():
        def _za(k, _):
            pltpu.sync_copy(zb, acc_sh.at[pl.ds(9600 + k * 32, 32), :])
            return 0
        lax.fori_loop(0, 12, _za, 0)
        pltpu.sync_copy(zb.at[pl.ds(0, 16), :], acc_sh.at[pl.ds(9984, 16), :])

    plsc.subcore_barrier()

    def _stage(c, rows, rs, cs, ns, gsem, ssem,
               orows, ors, ocs, ons, ogsem, ossem):
        @pl.when((lax.rem(c, 27) == 0) & (c < NCH))
        def _():
            base = wid * EPT + lax.div(c, 27) * 3456
            pltpu.sync_copy(row_h.at[pl.ds(base, 3456)], rowb)
            pltpu.sync_copy(col_h.at[pl.ds(base, 3456)], colb)
            pltpu.sync_copy(norm_h.at[pl.ds(base, 3456)], nrmb)

        @pl.when(c >= 2)
        def _():
            pltpu.make_async_copy(rows, acc_sh.at[cs], ssem).wait()

        @pl.when(c < NCH)
        def _():
            k = lax.rem(c, 27)

            def _cp(g, _):
                off = k * 128 + g * 16
                rs[pl.ds(g * 16, 16)] = rowb[pl.ds(off, 16)]
                cs[pl.ds(g * 16, 16)] = colb[pl.ds(off, 16)]
                ns[pl.ds(g * 16, 16)] = nrmb[pl.ds(off, 16)]
                return 0
            lax.fori_loop(0, 8, _cp, 0)
            pltpu.async_copy(v_h.at[rs], rows, gsem)

        @pl.when(c >= 1)
        def _():
            pltpu.make_async_copy(v_h.at[ors], orows, ogsem).wait()

            def _scale(g, _):
                n16 = ons[pl.ds(g * 16, 16)]
                for kk in range(16):
                    e = g * 16 + kk
                    s = n16[kk]
                    for j in range(8):
                        orows[e, pl.ds(j * 16, 16)] = (
                            orows[e, pl.ds(j * 16, 16)] * s)
                return 0
            lax.fori_loop(0, 8, _scale, 0)
            pltpu.async_copy(orows, acc_sh.at[ocs], ossem, add=True)

    def _iter(i, _):
        c0 = 2 * i
        _stage(c0, rows0, rs0, cs0, ns0, gsem0, ssem0,
               rows1, rs1, cs1, ns1, gsem1, ssem1)
        _stage(c0 + 1, rows1, rs1, cs1, ns1, gsem1, ssem1,
               rows0, rs0, cs0, ns0, gsem0, ssem0)
        return 0
    lax.fori_loop(0, (NCH + 1) // 2, _iter, 0)
    pltpu.make_async_copy(rows0, acc_sh.at[cs0], ssem0).wait()

    plsc.subcore_barrier()

    @pl.when(sid < 15)
    def _():
        pltpu.sync_copy(acc_sh.at[pl.ds(sid * 640, 640), :],
                        out_h.at[cid, pl.ds(sid * 640, 640), :])

    @pl.when(sid == 15)
    def _():
        pltpu.sync_copy(acc_sh.at[pl.ds(9600, 400), :],
                        out_h.at[cid, pl.ds(9600, 400), :])


_spmm_call = pl.kernel(
    _spmm_body,
    out_type=jax.ShapeDtypeStruct((2, N, D), jnp.float32),
    mesh=_MESH,
    scratch_types=[
        pltpu.VMEM_SHARED((N, D), jnp.float32),
        pltpu.VMEM((32, D), jnp.float32),
        pltpu.VMEM((3456,), jnp.int32),
        pltpu.VMEM((3456,), jnp.int32),
        pltpu.VMEM((3456,), jnp.float32),
        pltpu.VMEM((128, D), jnp.float32),
        pltpu.VMEM((128, D), jnp.float32),
        pltpu.VMEM((128,), jnp.int32),
        pltpu.VMEM((128,), jnp.int32),
        pltpu.VMEM((128,), jnp.int32),
        pltpu.VMEM((128,), jnp.int32),
        pltpu.VMEM((128,), jnp.float32),
        pltpu.VMEM((128,), jnp.float32),
        pltpu.SemaphoreType.DMA,
        pltpu.SemaphoreType.DMA,
        pltpu.SemaphoreType.DMA,
        pltpu.SemaphoreType.DMA,
    ],
    compiler_params=pltpu.CompilerParams(needs_layout_passes=False),
    name="mixhop_spmm_sc",
)


_BR = 1000
_GRID = N // _BR


def _dot_t(a, w):
    return lax.dot_general(a, w, (((1,), (1,)), ((), ())),
                           preferred_element_type=jnp.float32)


def _lin3_tc(x_ref, w0, w1, w2, b0, b1, b2, o0, o1, o2):
    x = x_ref[...]
    o0[...] = _dot_t(x, w0[...]) + b0[...]
    o1[...] = _dot_t(x, w1[...]) + b1[...]
    o2[...] = _dot_t(x, w2[...]) + b2[...]


def _add2_tc(a, b, o):
    o[...] = a[...] + b[...]


def _lin3b_tc(h00, p1a, p1b, p3a, p3b, g, bb, w0, w1, w2, b0, b1, b2,
              o0, o1, o2):
    hcat = jnp.concatenate(
        [h00[...], p1a[...] + p1b[...], p3a[...] + p3b[...]], axis=1)
    z = jnp.maximum(hcat * g[...] + bb[...], 0.0)
    o0[...] = _dot_t(z, w0[...]) + b0[...]
    o1[...] = _dot_t(z, w1[...]) + b1[...]
    o2[...] = _dot_t(z, w2[...]) + b2[...]


def _final_tc(t20, q1a, q1b, q3a, q3b, wf, bf, o):
    hcat = jnp.concatenate(
        [t20[...], q1a[...] + q1b[...], q3a[...] + q3b[...]], axis=1)
    o[...] = _dot_t(hcat, wf[...]) + bf[...]


def _row_spec(d):
    return pl.BlockSpec((_BR, d), lambda i: (i, 0))


def _full_spec(shape):
    return pl.BlockSpec(shape, lambda i: (0,) * len(shape))


def _lin3(x, w0, b0, w1, b1, w2, b2):
    outs = [jax.ShapeDtypeStruct((N, D), jnp.float32)] * 3
    din = x.shape[1]
    return pl.pallas_call(
        _lin3_tc,
        grid=(_GRID,),
        in_specs=[_row_spec(din)] + [_full_spec((D, din))] * 3
        + [_full_spec((1, D))] * 3,
        out_specs=[_row_spec(D)] * 3,
        out_shape=outs,
    )(x, w0, w1, w2, b0.reshape(1, D), b1.reshape(1, D), b2.reshape(1, D))


def _add2(a, b):
    return pl.pallas_call(
        _add2_tc,
        grid=(_GRID,),
        in_specs=[_row_spec(D)] * 2,
        out_specs=_row_spec(D),
        out_shape=jax.ShapeDtypeStruct((N, D), jnp.float32),
    )(a, b)


def _lin3b(h00, p1a, p1b, p3a, p3b, g, bb, w0, b0, w1, b1, w2, b2):
    outs = [jax.ShapeDtypeStruct((N, D), jnp.float32)] * 3
    return pl.pallas_call(
        _lin3b_tc,
        grid=(_GRID,),
        in_specs=[_row_spec(D)] * 5 + [_full_spec((1, 3 * D))] * 2
        + [_full_spec((D, 3 * D))] * 3 + [_full_spec((1, D))] * 3,
        out_specs=[_row_spec(D)] * 3,
        out_shape=outs,
    )(h00, p1a, p1b, p3a, p3b, g.reshape(1, 3 * D), bb.reshape(1, 3 * D),
      w0, w1, w2, b0.reshape(1, D), b1.reshape(1, D), b2.reshape(1, D))


def _final(t20, q1a, q1b, q3a, q3b, wf, bf):
    return pl.pallas_call(
        _final_tc,
        grid=(_GRID,),
        in_specs=[_row_spec(D)] * 5 + [_full_spec((D, 3 * D)),
                                       _full_spec((1, D))],
        out_specs=_row_spec(D),
        out_shape=jax.ShapeDtypeStruct((N, D), jnp.float32),
    )(t20, q1a, q1b, q3a, q3b, wf, bf.reshape(1, D))


def kernel(x, edge_index, edge_weight, W10, b10, W11, b11, W12, b12, g1, bb1,
           W20, b20, W21, b21, W22, b22, Wf, bf):
    loop = jnp.arange(N, dtype=jnp.int32)
    zpad = jnp.zeros((PAD,), jnp.int32)
    row_ext = jnp.concatenate([edge_index[0].astype(jnp.int32), loop, zpad])
    col_ext = jnp.concatenate([edge_index[1].astype(jnp.int32), loop, zpad])
    w_ext = jnp.concatenate([edge_weight.astype(jnp.float32),
                             jnp.ones((N,), jnp.float32),
                             jnp.zeros((PAD,), jnp.float32)])

    norm, _dis = _norm_call(row_ext, col_ext, w_ext)

    h00, t1, t2 = _lin3(x, W10, b10, W11, b11, W12, b12)
    p1 = _spmm_call(t1, row_ext, col_ext, norm)
    p2 = _spmm_call(t2, row_ext, col_ext, norm)
    u = _add2(p2[0], p2[1])
    p3 = _spmm_call(u, row_ext, col_ext, norm)

    t20, t21, t22 = _lin3b(h00, p1[0], p1[1], p3[0], p3[1],
                           g1, bb1, W20, b20, W21, b21, W22, b22)
    q1 = _spmm_call(t21, row_ext, col_ext, norm)
    q2 = _spmm_call(t22, row_ext, col_ext, norm)
    v2 = _add2(q2[0], q2[1])
    q3 = _spmm_call(v2, row_ext, col_ext, norm)

    return _final(t20, q1[0], q1[1], q3[0], q3[1], Wf, bf)


# speedup vs baseline: 17.7951x; 1.4404x over previous
import functools

import jax
import jax.numpy as jnp
from jax import lax
from jax.experimental import pallas as pl
from jax.experimental.pallas import tpu as pltpu
from jax.experimental.pallas import tpu_sc as plsc

N = 10000
E = 320000
D = 128
EP = 331776
PAD = EP - (E + N)
EPT = EP // 32
NCH = EPT // 128
EPT16 = EP // 16
NCH16 = EPT16 // 128

_MESH = plsc.VectorSubcoreMesh(
    core_axis_name="c", subcore_axis_name="s", num_cores=2, num_subcores=16)


def _norm_body(row_h, col_h, w_h, norm_h, dis_h,
               deg_sh, zb, colb, wbb, rowb, outb,
               cs0, cs1, ws0, ws1, deg_v, dis_v, sem0, sem1):
    cid = lax.axis_index("c")
    sid = lax.axis_index("s")
    wid = cid * 16 + sid

    def _z(i, _):
        zb[pl.ds(i * 16, 16)] = jnp.zeros((16,), jnp.float32)
        return 0
    lax.fori_loop(0, 40, _z, 0)

    @pl.when(sid < 15)
    def _():
        pltpu.sync_copy(zb, deg_sh.at[pl.ds(sid * 640, 640)])

    @pl.when(sid == 15)
    def _():
        pltpu.sync_copy(zb.at[pl.ds(0, 400)], deg_sh.at[pl.ds(9600, 400)])

    plsc.subcore_barrier()

    def _deg_stage(c, cs, ws, sem):
        @pl.when(lax.rem(c, 27) == 0)
        def _():
            base = sid * EPT16 + lax.div(c, 27) * 3456
            pltpu.sync_copy(col_h.at[pl.ds(base, 3456)], colb)
            pltpu.sync_copy(w_h.at[pl.ds(base, 3456)], wbb)

        @pl.when(c >= 2)
        def _():
            pltpu.make_async_copy(ws, deg_sh.at[cs], sem).wait()

        k = lax.rem(c, 27)

        def _cp(g, _):
            off = k * 128 + g * 16
            cs[pl.ds(g * 16, 16)] = colb[pl.ds(off, 16)]
            ws[pl.ds(g * 16, 16)] = wbb[pl.ds(off, 16)]
            return 0
        lax.fori_loop(0, 8, _cp, 0)
        pltpu.async_copy(ws, deg_sh.at[cs], sem, add=True)

    def _deg_iter(i, _):
        _deg_stage(2 * i, cs0, ws0, sem0)
        _deg_stage(2 * i + 1, cs1, ws1, sem1)
        return 0
    lax.fori_loop(0, NCH16 // 2, _deg_iter, 0)
    pltpu.make_async_copy(ws0, deg_sh.at[cs0], sem0).wait()
    pltpu.make_async_copy(ws1, deg_sh.at[cs1], sem1).wait()

    plsc.subcore_barrier()

    pltpu.sync_copy(deg_sh, deg_v)
    magic = jnp.int32(0x5F3759DF)

    def _rsqrt(i, _):
        d = deg_v[pl.ds(i * 16, 16)]
        yi = magic - lax.shift_right_logical(
            lax.bitcast_convert_type(d, jnp.int32), 1)
        y = lax.bitcast_convert_type(yi, jnp.float32)
        h = d * 0.5
        for _ in range(3):
            y = y * (1.5 - h * y * y)
        dis_v[pl.ds(i * 16, 16)] = y
        return 0
    lax.fori_loop(0, 625, _rsqrt, 0)

    for blk in range(3):
        base = wid * EPT + blk * 3456
        pltpu.sync_copy(row_h.at[pl.ds(base, 3456)], rowb)
        pltpu.sync_copy(col_h.at[pl.ds(base, 3456)], colb)
        pltpu.sync_copy(w_h.at[pl.ds(base, 3456)], wbb)

        def _g(g, _):
            r = rowb[pl.ds(g * 16, 16)]
            cc = colb[pl.ds(g * 16, 16)]
            nr = (plsc.load_gather(dis_v, [r]) * wbb[pl.ds(g * 16, 16)]
                  * plsc.load_gather(dis_v, [cc]))
            outb[pl.ds(g * 16, 16)] = nr
            return 0
        lax.fori_loop(0, 216, _g, 0)
        pltpu.sync_copy(outb, norm_h.at[pl.ds(base, 3456)])

    @pl.when(cid == 0)
    def _():
        @pl.when(sid < 15)
        def _():
            pltpu.sync_copy(dis_v.at[pl.ds(sid * 640, 640)],
                            dis_h.at[pl.ds(sid * 640, 640)])

        @pl.when(sid == 15)
        def _():
            pltpu.sync_copy(dis_v.at[pl.ds(9600, 400)],
                            dis_h.at[pl.ds(9600, 400)])


_norm_call = pl.kernel(
    _norm_body,
    out_type=(jax.ShapeDtypeStruct((EP,), jnp.float32),
              jax.ShapeDtypeStruct((N,), jnp.float32)),
    mesh=_MESH,
    scratch_types=[
        pltpu.VMEM_SHARED((N,), jnp.float32),
        pltpu.VMEM((640,), jnp.float32),
        pltpu.VMEM((3456,), jnp.int32),
        pltpu.VMEM((3456,), jnp.float32),
        pltpu.VMEM((3456,), jnp.int32),
        pltpu.VMEM((3456,), jnp.float32),
        pltpu.VMEM((128,), jnp.int32),
        pltpu.VMEM((128,), jnp.int32),
        pltpu.VMEM((128,), jnp.float32),
        pltpu.VMEM((128,), jnp.float32),
        pltpu.VMEM((N,), jnp.float32),
        pltpu.VMEM((N,), jnp.float32),
        pltpu.SemaphoreType.DMA,
        pltpu.SemaphoreType.DMA,
    ],
    compiler_params=pltpu.CompilerParams(needs_layout_passes=False),
    name="mixhop_norm_sc",
)


def _spmm_body(v_h, row_h, col_h, norm_h, out_h,
               acc_sh, zb, rowb, colb, nrmb,
               rows0, rows1, rs0, rs1, cs0, cs1, ns0, ns1,
               gsem0, gsem1, ssem0, ssem1):
    cid = lax.axis_index("c")
    sid = lax.axis_index("s")
    wid = cid * 16 + sid

    def _z(i, _):
        zb[lax.div(i, 8), pl.ds(lax.rem(i, 8) * 16, 16)] = (
            jnp.zeros((16,), jnp.float32))
        return 0
    lax.fori_loop(0, 256, _z, 0)

    @pl.when(sid < 15)
    def _():
        def _za(k, _):
            pltpu.async_copy(zb, acc_sh.at[pl.ds(sid * 640 + k * 32, 32), :],
                             gsem0)
            return 0
        lax.fori_loop(0, 20, _za, 0)

        def _zw(k, _):
            pltpu.make_async_copy(
                zb, acc_sh.at[pl.ds(sid * 640 + k * 32, 32), :],
                gsem0).wait()
            return 0
        lax.fori_loop(0, 20, _zw, 0)

    @pl.when(sid == 15)
    def _():
        def _za(k, _):
            pltpu.async_copy(zb, acc_sh.at[pl.ds(9600 + k * 32, 32), :],
                             gsem0)
            return 0
        lax.fori_loop(0, 12, _za, 0)
        pltpu.async_copy(zb.at[pl.ds(0, 16), :],
                         acc_sh.at[pl.ds(9984, 16), :], gsem0)

        def _zw(k, _):
            pltpu.make_async_copy(zb, acc_sh.at[pl.ds(9600 + k * 32, 32), :],
                                  gsem0).wait()
            return 0
        lax.fori_loop(0, 12, _zw, 0)
        pltpu.make_async_copy(zb.at[pl.ds(0, 16), :],
                              acc_sh.at[pl.ds(9984, 16), :], gsem0).wait()

    plsc.subcore_barrier()

    def _stage(c, rows, rs, cs, ns, gsem, ssem,
               orows, ors, ocs, ons, ogsem, ossem):
        @pl.when((lax.rem(c, 27) == 0) & (c < NCH))
        def _():
            base = wid * EPT + lax.div(c, 27) * 3456
            pltpu.sync_copy(row_h.at[pl.ds(base, 3456)], rowb)
            pltpu.sync_copy(col_h.at[pl.ds(base, 3456)], colb)
            pltpu.sync_copy(norm_h.at[pl.ds(base, 3456)], nrmb)

        @pl.when(c >= 2)
        def _():
            pltpu.make_async_copy(rows, acc_sh.at[cs], ssem).wait()

        @pl.when(c < NCH)
        def _():
            k = lax.rem(c, 27)

            def _cp(g, _):
                off = k * 128 + g * 16
                rs[pl.ds(g * 16, 16)] = rowb[pl.ds(off, 16)]
                cs[pl.ds(g * 16, 16)] = colb[pl.ds(off, 16)]
                ns[pl.ds(g * 16, 16)] = nrmb[pl.ds(off, 16)]
                return 0
            lax.fori_loop(0, 8, _cp, 0)
            pltpu.async_copy(v_h.at[rs], rows, gsem)

        @pl.when(c >= 1)
        def _():
            pltpu.make_async_copy(v_h.at[ors], orows, ogsem).wait()

            def _scale(g, _):
                n16 = ons[pl.ds(g * 16, 16)]
                for kk in range(16):
                    e = g * 16 + kk
                    s = n16[kk]
                    for j in range(8):
                        orows[e, pl.ds(j * 16, 16)] = (
                            orows[e, pl.ds(j * 16, 16)] * s)
                return 0
            lax.fori_loop(0, 8, _scale, 0)
            pltpu.async_copy(orows, acc_sh.at[ocs], ossem, add=True)

    def _iter(i, _):
        c0 = 2 * i
        _stage(c0, rows0, rs0, cs0, ns0, gsem0, ssem0,
               rows1, rs1, cs1, ns1, gsem1, ssem1)
        _stage(c0 + 1, rows1, rs1, cs1, ns1, gsem1, ssem1,
               rows0, rs0, cs0, ns0, gsem0, ssem0)
        return 0
    lax.fori_loop(0, (NCH + 1) // 2, _iter, 0)
    pltpu.make_async_copy(rows0, acc_sh.at[cs0], ssem0).wait()

    plsc.subcore_barrier()

    @pl.when(sid < 15)
    def _():
        pltpu.sync_copy(acc_sh.at[pl.ds(sid * 640, 640), :],
                        out_h.at[cid, pl.ds(sid * 640, 640), :])

    @pl.when(sid == 15)
    def _():
        pltpu.sync_copy(acc_sh.at[pl.ds(9600, 400), :],
                        out_h.at[cid, pl.ds(9600, 400), :])


_spmm_call = pl.kernel(
    _spmm_body,
    out_type=jax.ShapeDtypeStruct((2, N, D), jnp.float32),
    mesh=_MESH,
    scratch_types=[
        pltpu.VMEM_SHARED((N, D), jnp.float32),
        pltpu.VMEM((32, D), jnp.float32),
        pltpu.VMEM((3456,), jnp.int32),
        pltpu.VMEM((3456,), jnp.int32),
        pltpu.VMEM((3456,), jnp.float32),
        pltpu.VMEM((128, D), jnp.float32),
        pltpu.VMEM((128, D), jnp.float32),
        pltpu.VMEM((128,), jnp.int32),
        pltpu.VMEM((128,), jnp.int32),
        pltpu.VMEM((128,), jnp.int32),
        pltpu.VMEM((128,), jnp.int32),
        pltpu.VMEM((128,), jnp.float32),
        pltpu.VMEM((128,), jnp.float32),
        pltpu.SemaphoreType.DMA,
        pltpu.SemaphoreType.DMA,
        pltpu.SemaphoreType.DMA,
        pltpu.SemaphoreType.DMA,
    ],
    compiler_params=pltpu.CompilerParams(needs_layout_passes=False),
    name="mixhop_spmm_sc",
)


_BR = 1000
_GRID = N // _BR


def _dot_t(a, w):
    return lax.dot_general(a, w, (((1,), (1,)), ((), ())),
                           preferred_element_type=jnp.float32)


def _lin3_tc(x_ref, w0, w1, w2, b0, b1, b2, o0, o1, o2):
    x = x_ref[...]
    o0[...] = _dot_t(x, w0[...]) + b0[...]
    o1[...] = _dot_t(x, w1[...]) + b1[...]
    o2[...] = _dot_t(x, w2[...]) + b2[...]


def _add2_tc(a, b, o):
    o[...] = a[...] + b[...]


def _lin3b_tc(h00, p1a, p1b, p3a, p3b, g, bb, w0, w1, w2, b0, b1, b2,
              o0, o1, o2):
    hcat = jnp.concatenate(
        [h00[...], p1a[...] + p1b[...], p3a[...] + p3b[...]], axis=1)
    z = jnp.maximum(hcat * g[...] + bb[...], 0.0)
    o0[...] = _dot_t(z, w0[...]) + b0[...]
    o1[...] = _dot_t(z, w1[...]) + b1[...]
    o2[...] = _dot_t(z, w2[...]) + b2[...]


def _final_tc(t20, q1a, q1b, q3a, q3b, wf, bf, o):
    hcat = jnp.concatenate(
        [t20[...], q1a[...] + q1b[...], q3a[...] + q3b[...]], axis=1)
    o[...] = _dot_t(hcat, wf[...]) + bf[...]


def _row_spec(d):
    return pl.BlockSpec((_BR, d), lambda i: (i, 0))


def _full_spec(shape):
    return pl.BlockSpec(shape, lambda i: (0,) * len(shape))


def _lin3(x, w0, b0, w1, b1, w2, b2):
    outs = [jax.ShapeDtypeStruct((N, D), jnp.float32)] * 3
    din = x.shape[1]
    return pl.pallas_call(
        _lin3_tc,
        grid=(_GRID,),
        in_specs=[_row_spec(din)] + [_full_spec((D, din))] * 3
        + [_full_spec((1, D))] * 3,
        out_specs=[_row_spec(D)] * 3,
        out_shape=outs,
    )(x, w0, w1, w2, b0.reshape(1, D), b1.reshape(1, D), b2.reshape(1, D))


def _add2(a, b):
    return pl.pallas_call(
        _add2_tc,
        grid=(_GRID,),
        in_specs=[_row_spec(D)] * 2,
        out_specs=_row_spec(D),
        out_shape=jax.ShapeDtypeStruct((N, D), jnp.float32),
    )(a, b)


def _lin3b(h00, p1a, p1b, p3a, p3b, g, bb, w0, b0, w1, b1, w2, b2):
    outs = [jax.ShapeDtypeStruct((N, D), jnp.float32)] * 3
    return pl.pallas_call(
        _lin3b_tc,
        grid=(_GRID,),
        in_specs=[_row_spec(D)] * 5 + [_full_spec((1, 3 * D))] * 2
        + [_full_spec((D, 3 * D))] * 3 + [_full_spec((1, D))] * 3,
        out_specs=[_row_spec(D)] * 3,
        out_shape=outs,
    )(h00, p1a, p1b, p3a, p3b, g.reshape(1, 3 * D), bb.reshape(1, 3 * D),
      w0, w1, w2, b0.reshape(1, D), b1.reshape(1, D), b2.reshape(1, D))


def _final(t20, q1a, q1b, q3a, q3b, wf, bf):
    return pl.pallas_call(
        _final_tc,
        grid=(_GRID,),
        in_specs=[_row_spec(D)] * 5 + [_full_spec((D, 3 * D)),
                                       _full_spec((1, D))],
        out_specs=_row_spec(D),
        out_shape=jax.ShapeDtypeStruct((N, D), jnp.float32),
    )(t20, q1a, q1b, q3a, q3b, wf, bf.reshape(1, D))


def kernel(x, edge_index, edge_weight, W10, b10, W11, b11, W12, b12, g1, bb1,
           W20, b20, W21, b21, W22, b22, Wf, bf):
    loop = jnp.arange(N, dtype=jnp.int32)
    zpad = jnp.arange(PAD, dtype=jnp.int32)
    row_ext = jnp.concatenate([edge_index[0].astype(jnp.int32), loop, zpad])
    col_ext = jnp.concatenate([edge_index[1].astype(jnp.int32), loop, zpad])
    w_ext = jnp.concatenate([edge_weight.astype(jnp.float32),
                             jnp.ones((N,), jnp.float32),
                             jnp.zeros((PAD,), jnp.float32)])

    norm, _dis = _norm_call(row_ext, col_ext, w_ext)

    h00, t1, t2 = _lin3(x, W10, b10, W11, b11, W12, b12)
    p1 = _spmm_call(t1, row_ext, col_ext, norm)
    p2 = _spmm_call(t2, row_ext, col_ext, norm)
    u = _add2(p2[0], p2[1])
    p3 = _spmm_call(u, row_ext, col_ext, norm)

    t20, t21, t22 = _lin3b(h00, p1[0], p1[1], p3[0], p3[1],
                           g1, bb1, W20, b20, W21, b21, W22, b22)
    q1 = _spmm_call(t21, row_ext, col_ext, norm)
    q2 = _spmm_call(t22, row_ext, col_ext, norm)
    v2 = _add2(q2[0], q2[1])
    q3 = _spmm_call(v2, row_ext, col_ext, norm)

    return _final(t20, q1[0], q1[1], q3[0], q3[1], Wf, bf)


# speedup vs baseline: 18.0935x; 1.0168x over previous
import functools

import jax
import jax.numpy as jnp
from jax import lax
from jax.experimental import pallas as pl
from jax.experimental.pallas import tpu as pltpu
from jax.experimental.pallas import tpu_sc as plsc

N = 10000
E = 320000
D = 128
EP = 331776
PAD = EP - (E + N)
EPT = EP // 32
NCH = EPT // 128
EPT16 = EP // 16
NCH16 = EPT16 // 128

_MESH = plsc.VectorSubcoreMesh(
    core_axis_name="c", subcore_axis_name="s", num_cores=2, num_subcores=16)


def _norm_body(row_h, col_h, w_h, norm_h, dis_h,
               deg_sh, zb, colb, wbb, rowb, outb,
               cs0, cs1, ws0, ws1, deg_v, dis_v, sem0, sem1):
    cid = lax.axis_index("c")
    sid = lax.axis_index("s")
    wid = cid * 16 + sid

    def _z(i, _):
        zb[pl.ds(i * 16, 16)] = jnp.zeros((16,), jnp.float32)
        return 0
    lax.fori_loop(0, 40, _z, 0)

    @pl.when(sid < 15)
    def _():
        pltpu.sync_copy(zb, deg_sh.at[pl.ds(sid * 640, 640)])

    @pl.when(sid == 15)
    def _():
        pltpu.sync_copy(zb.at[pl.ds(0, 400)], deg_sh.at[pl.ds(9600, 400)])

    plsc.subcore_barrier()

    def _deg_stage(c, cs, ws, sem):
        @pl.when(lax.rem(c, 27) == 0)
        def _():
            base = sid * EPT16 + lax.div(c, 27) * 3456
            pltpu.sync_copy(col_h.at[pl.ds(base, 3456)], colb)
            pltpu.sync_copy(w_h.at[pl.ds(base, 3456)], wbb)

        @pl.when(c >= 2)
        def _():
            pltpu.make_async_copy(ws, deg_sh.at[cs], sem).wait()

        k = lax.rem(c, 27)

        def _cp(g, _):
            off = k * 128 + g * 16
            cs[pl.ds(g * 16, 16)] = colb[pl.ds(off, 16)]
            ws[pl.ds(g * 16, 16)] = wbb[pl.ds(off, 16)]
            return 0
        lax.fori_loop(0, 8, _cp, 0)
        pltpu.async_copy(ws, deg_sh.at[cs], sem, add=True)

    def _deg_iter(i, _):
        _deg_stage(2 * i, cs0, ws0, sem0)
        _deg_stage(2 * i + 1, cs1, ws1, sem1)
        return 0
    lax.fori_loop(0, NCH16 // 2, _deg_iter, 0)
    pltpu.make_async_copy(ws0, deg_sh.at[cs0], sem0).wait()
    pltpu.make_async_copy(ws1, deg_sh.at[cs1], sem1).wait()

    plsc.subcore_barrier()

    pltpu.sync_copy(deg_sh, deg_v)
    magic = jnp.int32(0x5F3759DF)

    def _rsqrt(i, _):
        d = deg_v[pl.ds(i * 16, 16)]
        yi = magic - lax.shift_right_logical(
            lax.bitcast_convert_type(d, jnp.int32), 1)
        y = lax.bitcast_convert_type(yi, jnp.float32)
        h = d * 0.5
        for _ in range(3):
            y = y * (1.5 - h * y * y)
        dis_v[pl.ds(i * 16, 16)] = y
        return 0
    lax.fori_loop(0, 625, _rsqrt, 0)

    for blk in range(3):
        base = wid * EPT + blk * 3456
        pltpu.sync_copy(row_h.at[pl.ds(base, 3456)], rowb)
        pltpu.sync_copy(col_h.at[pl.ds(base, 3456)], colb)
        pltpu.sync_copy(w_h.at[pl.ds(base, 3456)], wbb)

        def _g(g, _):
            r = rowb[pl.ds(g * 16, 16)]
            cc = colb[pl.ds(g * 16, 16)]
            nr = (plsc.load_gather(dis_v, [r]) * wbb[pl.ds(g * 16, 16)]
                  * plsc.load_gather(dis_v, [cc]))
            outb[pl.ds(g * 16, 16)] = nr
            return 0
        lax.fori_loop(0, 216, _g, 0)
        pltpu.sync_copy(outb, norm_h.at[pl.ds(base, 3456)])

    @pl.when(cid == 0)
    def _():
        @pl.when(sid < 15)
        def _():
            pltpu.sync_copy(dis_v.at[pl.ds(sid * 640, 640)],
                            dis_h.at[pl.ds(sid * 640, 640)])

        @pl.when(sid == 15)
        def _():
            pltpu.sync_copy(dis_v.at[pl.ds(9600, 400)],
                            dis_h.at[pl.ds(9600, 400)])


_norm_call = pl.kernel(
    _norm_body,
    out_type=(jax.ShapeDtypeStruct((EP,), jnp.float32),
              jax.ShapeDtypeStruct((N,), jnp.float32)),
    mesh=_MESH,
    scratch_types=[
        pltpu.VMEM_SHARED((N,), jnp.float32),
        pltpu.VMEM((640,), jnp.float32),
        pltpu.VMEM((3456,), jnp.int32),
        pltpu.VMEM((3456,), jnp.float32),
        pltpu.VMEM((3456,), jnp.int32),
        pltpu.VMEM((3456,), jnp.float32),
        pltpu.VMEM((128,), jnp.int32),
        pltpu.VMEM((128,), jnp.int32),
        pltpu.VMEM((128,), jnp.float32),
        pltpu.VMEM((128,), jnp.float32),
        pltpu.VMEM((N,), jnp.float32),
        pltpu.VMEM((N,), jnp.float32),
        pltpu.SemaphoreType.DMA,
        pltpu.SemaphoreType.DMA,
    ],
    compiler_params=pltpu.CompilerParams(needs_layout_passes=False),
    name="mixhop_norm_sc",
)


def _spmm_body(v_h, row_h, col_h, norm_h, out_h,
               acc_sh, zb, rowb, colb, nrmb,
               rows0, rows1, rs0, rs1, cs0, cs1, ns0, ns1,
               gsem0, gsem1, ssem0, ssem1, bsem):
    cid = lax.axis_index("c")
    sid = lax.axis_index("s")
    wid = cid * 16 + sid

    def _z(i, _):
        zb[lax.div(i, 8), pl.ds(lax.rem(i, 8) * 16, 16)] = (
            jnp.zeros((16,), jnp.float32))
        return 0
    lax.fori_loop(0, 256, _z, 0)

    @pl.when(sid < 15)
    def _():
        def _za(k, _):
            pltpu.async_copy(zb, acc_sh.at[pl.ds(sid * 640 + k * 32, 32), :],
                             gsem0)
            return 0
        lax.fori_loop(0, 20, _za, 0)

        def _zw(k, _):
            pltpu.make_async_copy(
                zb, acc_sh.at[pl.ds(sid * 640 + k * 32, 32), :],
                gsem0).wait()
            return 0
        lax.fori_loop(0, 20, _zw, 0)

    @pl.when(sid == 15)
    def _():
        def _za(k, _):
            pltpu.async_copy(zb, acc_sh.at[pl.ds(9600 + k * 32, 32), :],
                             gsem0)
            return 0
        lax.fori_loop(0, 12, _za, 0)
        pltpu.async_copy(zb.at[pl.ds(0, 16), :],
                         acc_sh.at[pl.ds(9984, 16), :], gsem0)

        def _zw(k, _):
            pltpu.make_async_copy(zb, acc_sh.at[pl.ds(9600 + k * 32, 32), :],
                                  gsem0).wait()
            return 0
        lax.fori_loop(0, 12, _zw, 0)
        pltpu.make_async_copy(zb.at[pl.ds(0, 16), :],
                              acc_sh.at[pl.ds(9984, 16), :], gsem0).wait()

    plsc.subcore_barrier()

    def _stage(c, rows, rs, cs, ns, gsem, ssem,
               orows, ors, ocs, ons, ogsem, ossem):
        @pl.when(c == 0)
        def _():
            base = wid * EPT
            pltpu.sync_copy(row_h.at[pl.ds(base, 3456)], rowb)
            pltpu.sync_copy(col_h.at[pl.ds(base, 3456)], colb)
            pltpu.sync_copy(norm_h.at[pl.ds(base, 3456)], nrmb)

        @pl.when((lax.rem(c, 27) == 0) & (c > 0) & (c < NCH))
        def _():
            base = wid * EPT + lax.div(c, 27) * 3456
            pltpu.make_async_copy(row_h.at[pl.ds(base, 3456)], rowb,
                                  bsem).wait()
            pltpu.make_async_copy(col_h.at[pl.ds(base, 3456)], colb,
                                  bsem).wait()
            pltpu.make_async_copy(norm_h.at[pl.ds(base, 3456)], nrmb,
                                  bsem).wait()

        @pl.when(c >= 2)
        def _():
            pltpu.make_async_copy(rows, acc_sh.at[cs], ssem).wait()

        @pl.when(c < NCH)
        def _():
            k = lax.rem(c, 27)

            def _cp(g, _):
                off = k * 128 + g * 16
                rs[pl.ds(g * 16, 16)] = rowb[pl.ds(off, 16)]
                cs[pl.ds(g * 16, 16)] = colb[pl.ds(off, 16)]
                ns[pl.ds(g * 16, 16)] = nrmb[pl.ds(off, 16)]
                return 0
            lax.fori_loop(0, 8, _cp, 0)
            pltpu.async_copy(v_h.at[rs], rows, gsem)

        @pl.when(lax.rem(c, 27) == 26)
        def _():
            @pl.when(c + 1 < NCH)
            def _():
                base = wid * EPT + lax.div(c + 1, 27) * 3456
                pltpu.async_copy(row_h.at[pl.ds(base, 3456)], rowb, bsem)
                pltpu.async_copy(col_h.at[pl.ds(base, 3456)], colb, bsem)
                pltpu.async_copy(norm_h.at[pl.ds(base, 3456)], nrmb, bsem)

        @pl.when(c >= 1)
        def _():
            pltpu.make_async_copy(v_h.at[ors], orows, ogsem).wait()

            def _scale(g, _):
                n16 = ons[pl.ds(g * 16, 16)]
                for kk in range(16):
                    e = g * 16 + kk
                    s = n16[kk]
                    for j in range(8):
                        orows[e, pl.ds(j * 16, 16)] = (
                            orows[e, pl.ds(j * 16, 16)] * s)
                return 0
            lax.fori_loop(0, 8, _scale, 0)
            pltpu.async_copy(orows, acc_sh.at[ocs], ossem, add=True)

    def _iter(i, _):
        c0 = 2 * i
        _stage(c0, rows0, rs0, cs0, ns0, gsem0, ssem0,
               rows1, rs1, cs1, ns1, gsem1, ssem1)
        _stage(c0 + 1, rows1, rs1, cs1, ns1, gsem1, ssem1,
               rows0, rs0, cs0, ns0, gsem0, ssem0)
        return 0
    lax.fori_loop(0, (NCH + 1) // 2, _iter, 0)
    pltpu.make_async_copy(rows0, acc_sh.at[cs0], ssem0).wait()

    plsc.subcore_barrier()

    @pl.when(sid < 15)
    def _():
        pltpu.sync_copy(acc_sh.at[pl.ds(sid * 640, 640), :],
                        out_h.at[cid, pl.ds(sid * 640, 640), :])

    @pl.when(sid == 15)
    def _():
        pltpu.sync_copy(acc_sh.at[pl.ds(9600, 400), :],
                        out_h.at[cid, pl.ds(9600, 400), :])


_spmm_call = pl.kernel(
    _spmm_body,
    out_type=jax.ShapeDtypeStruct((2, N, D), jnp.float32),
    mesh=_MESH,
    scratch_types=[
        pltpu.VMEM_SHARED((N, D), jnp.float32),
        pltpu.VMEM((32, D), jnp.float32),
        pltpu.VMEM((3456,), jnp.int32),
        pltpu.VMEM((3456,), jnp.int32),
        pltpu.VMEM((3456,), jnp.float32),
        pltpu.VMEM((128, D), jnp.float32),
        pltpu.VMEM((128, D), jnp.float32),
        pltpu.VMEM((128,), jnp.int32),
        pltpu.VMEM((128,), jnp.int32),
        pltpu.VMEM((128,), jnp.int32),
        pltpu.VMEM((128,), jnp.int32),
        pltpu.VMEM((128,), jnp.float32),
        pltpu.VMEM((128,), jnp.float32),
        pltpu.SemaphoreType.DMA,
        pltpu.SemaphoreType.DMA,
        pltpu.SemaphoreType.DMA,
        pltpu.SemaphoreType.DMA,
        pltpu.SemaphoreType.DMA,
    ],
    compiler_params=pltpu.CompilerParams(needs_layout_passes=False),
    name="mixhop_spmm_sc",
)


_BR = 1000
_GRID = N // _BR


def _dot_t(a, w):
    return lax.dot_general(a, w, (((1,), (1,)), ((), ())),
                           preferred_element_type=jnp.float32)


def _lin3_tc(x_ref, w0, w1, w2, b0, b1, b2, o0, o1, o2):
    x = x_ref[...]
    o0[...] = _dot_t(x, w0[...]) + b0[...]
    o1[...] = _dot_t(x, w1[...]) + b1[...]
    o2[...] = _dot_t(x, w2[...]) + b2[...]


def _add2_tc(a, b, o):
    o[...] = a[...] + b[...]


def _lin3b_tc(h00, p1a, p1b, p3a, p3b, g, bb, w0, w1, w2, b0, b1, b2,
              o0, o1, o2):
    hcat = jnp.concatenate(
        [h00[...], p1a[...] + p1b[...], p3a[...] + p3b[...]], axis=1)
    z = jnp.maximum(hcat * g[...] + bb[...], 0.0)
    o0[...] = _dot_t(z, w0[...]) + b0[...]
    o1[...] = _dot_t(z, w1[...]) + b1[...]
    o2[...] = _dot_t(z, w2[...]) + b2[...]


def _final_tc(t20, q1a, q1b, q3a, q3b, wf, bf, o):
    hcat = jnp.concatenate(
        [t20[...], q1a[...] + q1b[...], q3a[...] + q3b[...]], axis=1)
    o[...] = _dot_t(hcat, wf[...]) + bf[...]


def _row_spec(d):
    return pl.BlockSpec((_BR, d), lambda i: (i, 0))


def _full_spec(shape):
    return pl.BlockSpec(shape, lambda i: (0,) * len(shape))


def _lin3(x, w0, b0, w1, b1, w2, b2):
    outs = [jax.ShapeDtypeStruct((N, D), jnp.float32)] * 3
    din = x.shape[1]
    return pl.pallas_call(
        _lin3_tc,
        grid=(_GRID,),
        in_specs=[_row_spec(din)] + [_full_spec((D, din))] * 3
        + [_full_spec((1, D))] * 3,
        out_specs=[_row_spec(D)] * 3,
        out_shape=outs,
    )(x, w0, w1, w2, b0.reshape(1, D), b1.reshape(1, D), b2.reshape(1, D))


def _add2(a, b):
    return pl.pallas_call(
        _add2_tc,
        grid=(_GRID,),
        in_specs=[_row_spec(D)] * 2,
        out_specs=_row_spec(D),
        out_shape=jax.ShapeDtypeStruct((N, D), jnp.float32),
    )(a, b)


def _lin3b(h00, p1a, p1b, p3a, p3b, g, bb, w0, b0, w1, b1, w2, b2):
    outs = [jax.ShapeDtypeStruct((N, D), jnp.float32)] * 3
    return pl.pallas_call(
        _lin3b_tc,
        grid=(_GRID,),
        in_specs=[_row_spec(D)] * 5 + [_full_spec((1, 3 * D))] * 2
        + [_full_spec((D, 3 * D))] * 3 + [_full_spec((1, D))] * 3,
        out_specs=[_row_spec(D)] * 3,
        out_shape=outs,
    )(h00, p1a, p1b, p3a, p3b, g.reshape(1, 3 * D), bb.reshape(1, 3 * D),
      w0, w1, w2, b0.reshape(1, D), b1.reshape(1, D), b2.reshape(1, D))


def _final(t20, q1a, q1b, q3a, q3b, wf, bf):
    return pl.pallas_call(
        _final_tc,
        grid=(_GRID,),
        in_specs=[_row_spec(D)] * 5 + [_full_spec((D, 3 * D)),
                                       _full_spec((1, D))],
        out_specs=_row_spec(D),
        out_shape=jax.ShapeDtypeStruct((N, D), jnp.float32),
    )(t20, q1a, q1b, q3a, q3b, wf, bf.reshape(1, D))


def kernel(x, edge_index, edge_weight, W10, b10, W11, b11, W12, b12, g1, bb1,
           W20, b20, W21, b21, W22, b22, Wf, bf):
    loop = jnp.arange(N, dtype=jnp.int32)
    zpad = jnp.arange(PAD, dtype=jnp.int32)
    row_ext = jnp.concatenate([edge_index[0].astype(jnp.int32), loop, zpad])
    col_ext = jnp.concatenate([edge_index[1].astype(jnp.int32), loop, zpad])
    w_ext = jnp.concatenate([edge_weight.astype(jnp.float32),
                             jnp.ones((N,), jnp.float32),
                             jnp.zeros((PAD,), jnp.float32)])

    norm, _dis = _norm_call(row_ext, col_ext, w_ext)

    h00, t1, t2 = _lin3(x, W10, b10, W11, b11, W12, b12)
    p1 = _spmm_call(t1, row_ext, col_ext, norm)
    p2 = _spmm_call(t2, row_ext, col_ext, norm)
    u = _add2(p2[0], p2[1])
    p3 = _spmm_call(u, row_ext, col_ext, norm)

    t20, t21, t22 = _lin3b(h00, p1[0], p1[1], p3[0], p3[1],
                           g1, bb1, W20, b20, W21, b21, W22, b22)
    q1 = _spmm_call(t21, row_ext, col_ext, norm)
    q2 = _spmm_call(t22, row_ext, col_ext, norm)
    v2 = _add2(q2[0], q2[1])
    q3 = _spmm_call(v2, row_ext, col_ext, norm)

    return _final(t20, q1[0], q1[1], q3[0], q3[1], Wf, bf)
